# Initial kernel scaffold; baseline (speedup 1.0000x reference)
import jax
import jax.numpy as jnp
from jax.experimental import pallas as pl


def kernel(features, rois):
    raise NotImplementedError("write your pallas kernel here")


# speedup vs baseline: 44.6413x; 44.6413x over previous
import jax
import jax.numpy as jnp
from jax.experimental import pallas as pl

_POOL = 7
_SCALE = 0.0625
_S = 2
_N, _C, _H, _W = 4, 784, 32, 32
_CO = _C // (_POOL * _POOL)
_KB = 200
_BH = _N * _H
_CW = _CO * _POOL * _W


def _taps_1d(coord, valid_hi, base, iota, size):
    v = (coord >= -1.0) & (coord <= valid_hi)
    c = jnp.maximum(coord, 0.0)
    lo = jnp.minimum(jnp.floor(c), valid_hi - 1.0)
    hi = jnp.minimum(lo + 1.0, valid_hi - 1.0)
    c = jnp.where(lo >= valid_hi - 1.0, lo, c)
    l = c - lo
    h = 1.0 - l
    wl = jnp.where(v, h, 0.0)
    wh = jnp.where(v, l, 0.0)
    lo_i = (base + lo).astype(jnp.int32)
    hi_i = (base + hi).astype(jnp.int32)
    return (wl * (iota == lo_i).astype(jnp.float32)
            + wh * (iota == hi_i).astype(jnp.float32))


def _psroi_kernel(rois_ref, feat_ref, out_ref):
    ph = pl.program_id(0)
    rois = rois_ref[...]
    b = rois[:, 0:1]
    sw = rois[:, 1:2] * _SCALE - 0.5
    sh = rois[:, 2:3] * _SCALE - 0.5
    ew = rois[:, 3:4] * _SCALE - 0.5
    eh = rois[:, 4:5] * _SCALE - 0.5
    bin_w = jnp.maximum(ew - sw, 0.1) / _POOL
    bin_h = jnp.maximum(eh - sh, 0.1) / _POOL
    phf = ph.astype(jnp.float32)

    iota_bh = jax.lax.broadcasted_iota(jnp.int32, (_KB, _BH), 1)
    base_bh = b * _H
    ay = jnp.zeros((_KB, _BH), jnp.float32)
    for s in range(_S):
        y = sh + (phf + (s + 0.5) / _S) * bin_h
        ay = ay + _taps_1d(y, float(_H), base_bh, iota_bh, _BH)

    iota_w = jax.lax.broadcasted_iota(jnp.int32, (_KB, _POOL, _W), 2)
    pwf = jax.lax.broadcasted_iota(jnp.int32, (_KB, _POOL, 1), 1).astype(jnp.float32)
    ax = jnp.zeros((_KB, _POOL, _W), jnp.float32)
    for s in range(_S):
        x = sw[:, :, None] + (pwf + (s + 0.5) / _S) * bin_w[:, :, None]
        ax = ax + _taps_1d(x, float(_W), 0.0, iota_w, _W)

    t = jnp.dot(ay, feat_ref[0], preferred_element_type=jnp.float32)
    t = t.reshape(_KB, _CO, _POOL, _W)
    out = (t * ax[:, None, :, :]).sum(axis=3) * 0.25
    out_ref[...] = out[None]


def kernel(features, rois):
    n, c, hh, ww = features.shape
    k = rois.shape[0]
    f = features.reshape(n, _CO, _POOL, _POOL, hh, ww)
    f = f.transpose(2, 0, 4, 1, 3, 5).reshape(_POOL, _BH, _CW)
    out = pl.pallas_call(
        _psroi_kernel,
        grid=(_POOL, k // _KB),
        in_specs=[
            pl.BlockSpec((_KB, 5), lambda ph, kb: (kb, 0)),
            pl.BlockSpec((1, _BH, _CW), lambda ph, kb: (ph, 0, 0)),
        ],
        out_specs=pl.BlockSpec((1, _KB, _CO, _POOL), lambda ph, kb: (ph, kb, 0, 0)),
        out_shape=jax.ShapeDtypeStruct((_POOL, k, _CO, _POOL), jnp.float32),
    )(rois, f)
    return out.transpose(1, 2, 0, 3)


# speedup vs baseline: 132.6776x; 2.9721x over previous
import jax
import jax.numpy as jnp
from jax.experimental import pallas as pl

_POOL = 7
_SCALE = 0.0625
_S = 2
_N, _C, _H, _W = 4, 784, 32, 32
_CO = _C // (_POOL * _POOL)
_KB = 200
_BH = _N * _H
_CW = _CO * _POOL * _W


def _psroi_kernel(rois_ref, feat_ref, out_ref):
    ph = pl.program_id(0)
    rois = rois_ref[0]
    b = rois[0:1, :]
    sw = rois[1:2, :] * _SCALE - 0.5
    sh = rois[2:3, :] * _SCALE - 0.5
    ew = rois[3:4, :] * _SCALE - 0.5
    eh = rois[4:5, :] * _SCALE - 0.5
    bin_w = jnp.maximum(ew - sw, 0.1) / _POOL
    bin_h = jnp.maximum(eh - sh, 0.1) / _POOL
    phf = ph.astype(jnp.float32)

    hrel = (jax.lax.broadcasted_iota(jnp.int32, (_BH, _KB), 0).astype(jnp.float32)
            - b * _H)
    ayt = jnp.zeros((_BH, _KB), jnp.float32)
    for s in range(_S):
        y = sh + (phf + (s + 0.5) / _S) * bin_h
        vy = (y >= -1.0) & (y <= _H)
        yc = jnp.minimum(jnp.maximum(y, 0.0), _H - 1.0)
        w = jnp.maximum(1.0 - jnp.abs(yc - hrel), 0.0)
        ayt = ayt + jnp.where(vy, w, 0.0)

    iota_pww = jax.lax.broadcasted_iota(jnp.int32, (_POOL * _W, _KB), 0)
    wrel = (iota_pww & (_W - 1)).astype(jnp.float32)
    pwf = (iota_pww >> 5).astype(jnp.float32)
    axt = jnp.zeros((_POOL * _W, _KB), jnp.float32)
    for s in range(_S):
        x = sw + (pwf + (s + 0.5) / _S) * bin_w
        vx = (x >= -1.0) & (x <= _W)
        xc = jnp.minimum(jnp.maximum(x, 0.0), _W - 1.0)
        w = jnp.maximum(1.0 - jnp.abs(xc - wrel), 0.0)
        axt = axt + jnp.where(vx, w, 0.0)
    axt = axt * 0.25

    t = jnp.dot(feat_ref[0], ayt, preferred_element_type=jnp.float32)
    t = t.reshape(_CO, _POOL, _W, _KB) * axt.reshape(_POOL, _W, _KB)[None]
    out_ref[...] = t.sum(axis=2)[None, None]


def kernel(features, rois):
    n, c, hh, ww = features.shape
    k = rois.shape[0]
    f = features.reshape(n, _CO, _POOL, _POOL, hh, ww)
    f = f.transpose(2, 1, 3, 5, 0, 4).reshape(_POOL, _CW, _BH)
    rois_t = rois.reshape(k // _KB, _KB, 5).transpose(0, 2, 1)
    out = pl.pallas_call(
        _psroi_kernel,
        grid=(_POOL, k // _KB),
        in_specs=[
            pl.BlockSpec((1, 5, _KB), lambda ph, kb: (kb, 0, 0)),
            pl.BlockSpec((1, _CW, _BH), lambda ph, kb: (ph, 0, 0)),
        ],
        out_specs=pl.BlockSpec((1, 1, _CO, _POOL, _KB),
                               lambda ph, kb: (kb, ph, 0, 0, 0)),
        out_shape=jax.ShapeDtypeStruct((k // _KB, _POOL, _CO, _POOL, _KB),
                                       jnp.float32),
    )(rois_t, f)
    return out.transpose(0, 4, 2, 1, 3).reshape(k, _CO, _POOL, _POOL)


# speedup vs baseline: 156.2845x; 1.1779x over previous
import jax
import jax.numpy as jnp
from jax.experimental import pallas as pl

_POOL = 7
_SCALE = 0.0625
_S = 2
_N, _C, _H, _W = 4, 784, 32, 32
_CO = _C // (_POOL * _POOL)
_KB = 1000
_BH = _N * _H
_CW = _CO * _POOL * _W


def _psroi_kernel(rois_ref, feat_ref, out_ref):
    ph = pl.program_id(0)
    rois = rois_ref[...]
    b = rois[0:1, :]
    sw = rois[1:2, :] * _SCALE - 0.5
    sh = rois[2:3, :] * _SCALE - 0.5
    ew = rois[3:4, :] * _SCALE - 0.5
    eh = rois[4:5, :] * _SCALE - 0.5
    bin_w = jnp.maximum(ew - sw, 0.1) / _POOL
    bin_h = jnp.maximum(eh - sh, 0.1) / _POOL
    phf = ph.astype(jnp.float32)

    hrel = (jax.lax.broadcasted_iota(jnp.int32, (_BH, _KB), 0).astype(jnp.float32)
            - b * _H)
    ayt = jnp.zeros((_BH, _KB), jnp.float32)
    for s in range(_S):
        y = sh + (phf + (s + 0.5) / _S) * bin_h
        vy = (y >= -1.0) & (y <= _H)
        yc = jnp.minimum(jnp.maximum(y, 0.0), _H - 1.0)
        w = jnp.maximum(1.0 - jnp.abs(yc - hrel), 0.0)
        ayt = ayt + jnp.where(vy, w, 0.0)

    iota_pww = jax.lax.broadcasted_iota(jnp.int32, (_POOL * _W, _KB), 0)
    wrel = (iota_pww & (_W - 1)).astype(jnp.float32)
    pwf = (iota_pww >> 5).astype(jnp.float32)
    axt = jnp.zeros((_POOL * _W, _KB), jnp.float32)
    for s in range(_S):
        x = sw + (pwf + (s + 0.5) / _S) * bin_w
        vx = (x >= -1.0) & (x <= _W)
        xc = jnp.minimum(jnp.maximum(x, 0.0), _W - 1.0)
        w = jnp.maximum(1.0 - jnp.abs(xc - wrel), 0.0)
        axt = axt + jnp.where(vx, w, 0.0)
    axt = axt * 0.25

    t = jnp.dot(feat_ref[0], ayt, preferred_element_type=jnp.float32)
    t = t.reshape(_CO, _POOL, _W, _KB) * axt.reshape(_POOL, _W, _KB)[None]
    out_ref[...] = t.sum(axis=2)[None]


def kernel(features, rois):
    n, c, hh, ww = features.shape
    k = rois.shape[0]
    f = features.reshape(n, _CO, _POOL, _POOL, hh, ww)
    f = f.transpose(2, 1, 3, 5, 0, 4).reshape(_POOL, _CW, _BH)
    rois_t = rois.T
    out = pl.pallas_call(
        _psroi_kernel,
        grid=(_POOL,),
        in_specs=[
            pl.BlockSpec((5, _KB), lambda ph: (0, 0)),
            pl.BlockSpec((1, _CW, _BH), lambda ph: (ph, 0, 0)),
        ],
        out_specs=pl.BlockSpec((1, _CO, _POOL, _KB), lambda ph: (ph, 0, 0, 0)),
        out_shape=jax.ShapeDtypeStruct((_POOL, _CO, _POOL, k), jnp.float32),
    )(rois_t, f)
    return out.transpose(3, 1, 0, 2)


# speedup vs baseline: 284.1709x; 1.8183x over previous
import jax
import jax.numpy as jnp
from jax.experimental import pallas as pl

_POOL = 7
_SCALE = 0.0625
_S = 2
_N, _C, _FH, _FW = 4, 784, 32, 32
_CO = _C // (_POOL * _POOL)
_RW = _CO * _POOL * _FH
_BW = _N * _FW


def _psroi_kernel(rois_ref, feat_ref, out_ref):
    pw = pl.program_id(0)
    rois = rois_ref[...]
    kk = rois.shape[1]
    b = rois[0:1, :]
    sw = rois[1:2, :] * _SCALE - 0.5
    sh = rois[2:3, :] * _SCALE - 0.5
    ew = rois[3:4, :] * _SCALE - 0.5
    eh = rois[4:5, :] * _SCALE - 0.5
    bin_w = jnp.maximum(ew - sw, 0.1) / _POOL
    bin_h = jnp.maximum(eh - sh, 0.1) / _POOL
    pwf = pw.astype(jnp.float32)

    iota_bw = jax.lax.broadcasted_iota(jnp.int32, (_BW, kk), 0)
    wrel = (iota_bw & (_FW - 1)).astype(jnp.float32)
    brow = (iota_bw >> 5).astype(jnp.float32)
    axt = jnp.zeros((_BW, kk), jnp.float32)
    for s in range(_S):
        x = sw + (pwf + (s + 0.5) / _S) * bin_w
        vx = (x >= -1.0) & (x <= _FW)
        xc = jnp.minimum(jnp.maximum(x, 0.0), _FW - 1.0)
        axt = axt + jnp.where(vx, jnp.maximum(1.0 - jnp.abs(xc - wrel), 0.0), 0.0)
    axt = jnp.where(brow == b, axt * 0.25, 0.0)

    iota_phh = jax.lax.broadcasted_iota(jnp.int32, (_POOL * _FH, kk), 0)
    hrel = (iota_phh & (_FH - 1)).astype(jnp.float32)
    phrow = (iota_phh >> 5).astype(jnp.float32)
    ayt = jnp.zeros((_POOL * _FH, kk), jnp.float32)
    for s in range(_S):
        y = sh + (phrow + (s + 0.5) / _S) * bin_h
        vy = (y >= -1.0) & (y <= _FH)
        yc = jnp.minimum(jnp.maximum(y, 0.0), _FH - 1.0)
        ayt = ayt + jnp.where(vy, jnp.maximum(1.0 - jnp.abs(yc - hrel), 0.0), 0.0)

    fm = jnp.concatenate(
        [feat_ref[i, :, :, 0].reshape(_RW, _FW) for i in range(_N)], axis=1)

    t = jnp.dot(fm, axt, preferred_element_type=jnp.float32)
    t = t.reshape(_CO, _POOL, _FH, kk) * ayt.reshape(_POOL, _FH, kk)[None]
    out_ref[...] = t.sum(axis=2)[None]


def kernel(features, rois):
    n, c, hh, ww = features.shape
    k = rois.shape[0]
    f = features.reshape(n, _CO, _POOL, _POOL, hh, ww)
    rois_t = rois.T
    out = pl.pallas_call(
        _psroi_kernel,
        grid=(_POOL,),
        in_specs=[
            pl.BlockSpec((5, k), lambda pw: (0, 0)),
            pl.BlockSpec((_N, _CO, _POOL, 1, hh, ww), lambda pw: (0, 0, 0, pw, 0, 0)),
        ],
        out_specs=pl.BlockSpec((1, _CO, _POOL, k), lambda pw: (pw, 0, 0, 0)),
        out_shape=jax.ShapeDtypeStruct((_POOL, _CO, _POOL, k), jnp.float32),
    )(rois_t, f)
    return out.transpose(3, 1, 2, 0)


# speedup vs baseline: 294.2692x; 1.0355x over previous
import jax
import jax.numpy as jnp
from jax.experimental import pallas as pl

_POOL = 7
_SCALE = 0.0625
_S = 2
_N, _C, _FH, _FW = 4, 784, 32, 32
_CO = _C // (_POOL * _POOL)
_RW = _CO * _POOL * _FH
_BW = _N * _FW


def _psroi_kernel(rois_ref, feat_ref, out_ref):
    pw = pl.program_id(0)
    rois = rois_ref[...]
    kk = rois.shape[1]
    b = rois[0:1, :]
    sw = rois[1:2, :] * _SCALE - 0.5
    sh = rois[2:3, :] * _SCALE - 0.5
    ew = rois[3:4, :] * _SCALE - 0.5
    eh = rois[4:5, :] * _SCALE - 0.5
    bin_w = jnp.maximum(ew - sw, 0.1) / _POOL
    bin_h = jnp.maximum(eh - sh, 0.1) / _POOL
    pwf = pw.astype(jnp.float32)

    iota_bw = jax.lax.broadcasted_iota(jnp.int32, (_BW, kk), 0)
    wrel = (iota_bw & (_FW - 1)).astype(jnp.float32)
    brow = (iota_bw >> 5).astype(jnp.float32)
    axt = jnp.zeros((_BW, kk), jnp.float32)
    for s in range(_S):
        x = sw + (pwf + (s + 0.5) / _S) * bin_w
        vx = (x >= -1.0) & (x <= _FW)
        xc = jnp.minimum(jnp.maximum(x, 0.0), _FW - 1.0)
        axt = axt + jnp.where(vx, jnp.maximum(1.0 - jnp.abs(xc - wrel), 0.0), 0.0)
    axt = jnp.where(brow == b, axt * 0.25, 0.0)
    axt = axt.astype(jnp.bfloat16)

    iota_phh = jax.lax.broadcasted_iota(jnp.int32, (_POOL * _FH, kk), 0)
    hrel = (iota_phh & (_FH - 1)).astype(jnp.float32)
    phrow = (iota_phh >> 5).astype(jnp.float32)
    ayt = jnp.zeros((_POOL * _FH, kk), jnp.float32)
    for s in range(_S):
        y = sh + (phrow + (s + 0.5) / _S) * bin_h
        vy = (y >= -1.0) & (y <= _FH)
        yc = jnp.minimum(jnp.maximum(y, 0.0), _FH - 1.0)
        ayt = ayt + jnp.where(vy, jnp.maximum(1.0 - jnp.abs(yc - hrel), 0.0), 0.0)

    fm = jnp.concatenate(
        [feat_ref[i, :, :, 0].reshape(_RW, _FW) for i in range(_N)], axis=1)

    t = jnp.dot(fm, axt, preferred_element_type=jnp.float32)
    t = t.reshape(_CO, _POOL, _FH, kk) * ayt.reshape(_POOL, _FH, kk)[None]
    out_ref[...] = t.sum(axis=2)[None]


def kernel(features, rois):
    n, c, hh, ww = features.shape
    k = rois.shape[0]
    f = features.astype(jnp.bfloat16).reshape(n, _CO, _POOL, _POOL, hh, ww)
    rois_t = rois.T
    out = pl.pallas_call(
        _psroi_kernel,
        grid=(_POOL,),
        in_specs=[
            pl.BlockSpec((5, k), lambda pw: (0, 0)),
            pl.BlockSpec((_N, _CO, _POOL, 1, hh, ww), lambda pw: (0, 0, 0, pw, 0, 0)),
        ],
        out_specs=pl.BlockSpec((1, _CO, _POOL, k), lambda pw: (pw, 0, 0, 0)),
        out_shape=jax.ShapeDtypeStruct((_POOL, _CO, _POOL, k), jnp.float32),
    )(rois_t, f)
    return out.transpose(3, 1, 2, 0)
